# Initial kernel scaffold; baseline (speedup 1.0000x reference)
import jax
import jax.numpy as jnp
from jax.experimental import pallas as pl


def kernel(node_features, edge_node_indices, edge_features, fe_W, fe_b, fh_W, fh_b):
    raise NotImplementedError("write your pallas kernel here")


# speedup vs baseline: 3.2496x; 3.2496x over previous
import functools

import jax
import jax.numpy as jnp
from jax import lax
from jax.experimental import pallas as pl
from jax.experimental.pallas import tpu as pltpu
from jax.experimental.pallas import tpu_sc as plsc

NN = 10000
NE = 320000
D = 128
DE = 16
M = 4
NNP = 10240
NEP = 327680
NWK = 32
EPW = NEP // NWK
CS = 512
NCH = EPW // CS
ACCR = NNP * M // 128

f32 = jnp.float32
i32 = jnp.int32


def _stage_a1_body(nf_ref, *refs):
    w_refs, out_refs = refs[:2 * M], refs[2 * M:]
    nfa = nf_ref[...]
    for c in range(2 * M):
        out_refs[c][...] = lax.squeeze(
            lax.dot_general(nfa, w_refs[c][...], (((1,), (0,)), ((), ())),
                            preferred_element_type=f32), [1])


def _stage_a1(nfa, wcols):
    return pl.pallas_call(
        _stage_a1_body,
        grid=(1,),
        in_specs=[pl.BlockSpec((NNP, D + 1), lambda i: (0, 0))]
        + [pl.BlockSpec((D + 1, 1), lambda i: (0, 0))] * (2 * M),
        out_specs=[pl.BlockSpec((NNP,), lambda i: (0,))] * (2 * M),
        out_shape=[jax.ShapeDtypeStruct((NNP,), f32)] * (2 * M),
    )(nfa, *wcols)


def _stage_a2_body(ef_ref, *refs):
    w_refs, out_refs = refs[:M], refs[M:]
    efa = ef_ref[...]
    for c in range(M):
        out_refs[c][...] = lax.squeeze(
            lax.dot_general(efa, w_refs[c][...], (((1,), (0,)), ((), ())),
                            preferred_element_type=f32), [1])


def _stage_a2(efp, w2cols):
    grid = 160
    eb = NEP // grid
    return pl.pallas_call(
        _stage_a2_body,
        grid=(grid,),
        in_specs=[pl.BlockSpec((eb, DE), lambda i: (i, 0))]
        + [pl.BlockSpec((DE, 1), lambda i: (0, 0))] * M,
        out_specs=[pl.BlockSpec((eb,), lambda i: (i,))] * M,
        out_shape=[jax.ShapeDtypeStruct((NEP,), f32)] * M,
    )(efp, *w2cols)


def _sc_messages(pcs, ecs, i0, i1):
    mesh = plsc.VectorSubcoreMesh(core_axis_name="c", subcore_axis_name="s")

    @functools.partial(
        pl.kernel,
        mesh=mesh,
        compiler_params=pltpu.CompilerParams(needs_layout_passes=False),
        out_type=jax.ShapeDtypeStruct((NWK, NNP * M), f32),
        scratch_types=(
            [pltpu.VMEM((NNP,), f32)] * (2 * M)
            + [pltpu.VMEM((CS,), f32)] * M
            + [
                pltpu.VMEM((CS,), i32),
                pltpu.VMEM((CS,), i32),
                pltpu.VMEM((NNP * M,), f32),
            ]
        ),
    )
    def k(p0_hbm, p1_hbm, p2_hbm, p3_hbm, p4_hbm, p5_hbm, p6_hbm, p7_hbm,
          e0_hbm, e1_hbm, e2_hbm, e3_hbm, i0_hbm, i1_hbm, out_hbm,
          p0_v, p1_v, p2_v, p3_v, p4_v, p5_v, p6_v, p7_v,
          e0_v, e1_v, e2_v, e3_v, i0_v, i1_v, acc_v):
        p_hbm = [p0_hbm, p1_hbm, p2_hbm, p3_hbm, p4_hbm, p5_hbm, p6_hbm,
                 p7_hbm]
        e_hbm = [e0_hbm, e1_hbm, e2_hbm, e3_hbm]
        p_v = [p0_v, p1_v, p2_v, p3_v, p4_v, p5_v, p6_v, p7_v]
        e_v = [e0_v, e1_v, e2_v, e3_v]

        cid = lax.axis_index("c")
        sid = lax.axis_index("s")
        wid = cid * 16 + sid

        zero16 = jnp.zeros((16,), f32)

        def zero_body(r, c2):
            for c in range(8):
                acc_v[pl.ds(r * 128 + c * 16, 16)] = zero16
            return c2

        lax.fori_loop(0, NNP * M // 128, zero_body, 0)
        for c in range(2 * M):
            pltpu.sync_copy(p_hbm[c], p_v[c])

        def group_body(g, c2):
            off = g * 16
            iv0 = i0_v[pl.ds(off, 16)]
            iv1 = i1_v[pl.ds(off, 16)]
            ps = []
            for c in range(M):
                p = (plsc.load_gather(p_v[c], [iv0])
                     + plsc.load_gather(p_v[c + M], [iv1])
                     + e_v[c][pl.ds(off, 16)])
                ps.append(jnp.where(p >= 0.0, p, p * 0.01))
            mx = jnp.maximum(jnp.maximum(ps[0], ps[1]),
                             jnp.maximum(ps[2], ps[3]))
            es = [jnp.exp(p - mx) for p in ps]
            inv = 1.0 / (es[0] + es[1] + es[2] + es[3])
            n4 = iv0 * M
            for c in range(M):
                plsc.addupdate_scatter(acc_v, [n4 + c], es[c] * inv)
            return c2

        def chunk_body(t, carry):
            base = wid * EPW + t * CS
            pltpu.sync_copy(i0_hbm.at[pl.ds(base, CS)], i0_v)
            pltpu.sync_copy(i1_hbm.at[pl.ds(base, CS)], i1_v)
            for c in range(M):
                pltpu.sync_copy(e_hbm[c].at[pl.ds(base, CS)], e_v[c])
            lax.fori_loop(0, CS // 16, group_body, 0)
            return carry

        lax.fori_loop(0, NCH, chunk_body, 0)

        pltpu.sync_copy(acc_v, out_hbm.at[wid])

    return k(*pcs, *ecs, i0, i1)


def _stage_r_body(acc_ref, out_ref):
    out_ref[...] = jnp.sum(acc_ref[...], axis=0)


def _stage_r(acc):
    grid = 5
    rb = ACCR // grid
    return pl.pallas_call(
        _stage_r_body,
        grid=(grid,),
        in_specs=[pl.BlockSpec((NWK, rb, 128), lambda i: (0, i, 0))],
        out_specs=pl.BlockSpec((rb, 128), lambda i: (i, 0)),
        out_shape=jax.ShapeDtypeStruct((ACCR, 128), f32),
    )(acc)


def _stage_c_body(nf_ref, acc_ref, wn_ref, wm_ref, b_ref, z_ref):
    a = acc_ref[...]
    z_ref[...] = (
        lax.dot_general(nf_ref[...], wn_ref[...], (((1,), (0,)), ((), ())),
                        preferred_element_type=f32)
        + lax.dot_general(a, wm_ref[...], (((1,), (0,)), ((), ())),
                          preferred_element_type=f32)
        + b_ref[...])


def _stage_c(nf, msum, wn, wm, b):
    grid = 5
    nb = 2048
    return pl.pallas_call(
        _stage_c_body,
        grid=(grid,),
        in_specs=[
            pl.BlockSpec((nb, D), lambda i: (i, 0)),
            pl.BlockSpec((nb, M), lambda i: (i, 0)),
            pl.BlockSpec((D, D), lambda i: (0, 0)),
            pl.BlockSpec((M, D), lambda i: (0, 0)),
            pl.BlockSpec((1, D), lambda i: (0, 0)),
        ],
        out_specs=pl.BlockSpec((nb, D), lambda i: (i, 0)),
        out_shape=jax.ShapeDtypeStruct((NN, D), f32),
    )(nf, msum, wn, wm, b)


def kernel(node_features, edge_node_indices, edge_features, fe_W, fe_b, fh_W, fh_b):
    nf = node_features.astype(f32)
    idx0 = jnp.concatenate([edge_node_indices[0].astype(i32),
                            jnp.full((NEP - NE,), NN, i32)])
    idx1 = jnp.concatenate([edge_node_indices[1].astype(i32),
                            jnp.zeros((NEP - NE,), i32)])
    ef = edge_features.astype(f32)

    wcat = jnp.concatenate([fe_W[:D], fe_W[D:2 * D]], axis=1)
    brow = jnp.concatenate([fe_b, jnp.zeros((M,), f32)]).reshape(1, 2 * M)
    wcat_aug = jnp.concatenate([wcat, brow], axis=0)
    nfa = jnp.pad(jnp.concatenate([nf, jnp.ones((NN, 1), f32)], axis=1),
                  ((0, NNP - NN), (0, 0)))
    wcols = [wcat_aug[:, c:c + 1] for c in range(2 * M)]
    w2cols = [fe_W[2 * D:, c:c + 1] for c in range(M)]

    efp = jnp.pad(ef, ((0, NEP - NE), (0, 0)))

    pcs = _stage_a1(nfa, wcols)
    ecs = _stage_a2(efp, w2cols)

    acc32 = _sc_messages(pcs, ecs, idx0, idx1).reshape(NWK, ACCR, 128)
    msum = _stage_r(acc32).reshape(NNP, M)[:NN]

    return _stage_c(nf, msum, fh_W[:D], fh_W[D:], fh_b.reshape(1, D))


# speedup vs baseline: 3.3443x; 1.0291x over previous
import functools

import jax
import jax.numpy as jnp
from jax import lax
from jax.experimental import pallas as pl
from jax.experimental.pallas import tpu as pltpu
from jax.experimental.pallas import tpu_sc as plsc

NN = 10000
NE = 320000
D = 128
DE = 16
M = 4
NNP = 10240
NEP = 327680
NWK = 32
EPW = NEP // NWK
CS = 1024
NCH = EPW // CS
ACCR = NNP * M // 128

f32 = jnp.float32
i32 = jnp.int32


def _stage_a1_body(nf_ref, *refs):
    w_refs, out_refs = refs[:2 * M], refs[2 * M:]
    nfa = nf_ref[...]
    for c in range(2 * M):
        out_refs[c][...] = lax.squeeze(
            lax.dot_general(nfa, w_refs[c][...], (((1,), (0,)), ((), ())),
                            preferred_element_type=f32), [1])


def _stage_a1(nfa, wcols):
    return pl.pallas_call(
        _stage_a1_body,
        grid=(1,),
        in_specs=[pl.BlockSpec((NNP, D + 1), lambda i: (0, 0))]
        + [pl.BlockSpec((D + 1, 1), lambda i: (0, 0))] * (2 * M),
        out_specs=[pl.BlockSpec((NNP,), lambda i: (0,))] * (2 * M),
        out_shape=[jax.ShapeDtypeStruct((NNP,), f32)] * (2 * M),
    )(nfa, *wcols)


def _stage_a2_body(ef_ref, *refs):
    w_refs, out_refs = refs[:M], refs[M:]
    efa = ef_ref[...]
    for c in range(M):
        out_refs[c][...] = lax.squeeze(
            lax.dot_general(efa, w_refs[c][...], (((1,), (0,)), ((), ())),
                            preferred_element_type=f32), [1])


def _stage_a2(efp, w2cols):
    grid = 160
    eb = NEP // grid
    return pl.pallas_call(
        _stage_a2_body,
        grid=(grid,),
        in_specs=[pl.BlockSpec((eb, DE), lambda i: (i, 0))]
        + [pl.BlockSpec((DE, 1), lambda i: (0, 0))] * M,
        out_specs=[pl.BlockSpec((eb,), lambda i: (i,))] * M,
        out_shape=[jax.ShapeDtypeStruct((NEP,), f32)] * M,
    )(efp, *w2cols)


def _sc_messages(pcs, ecs, i0, i1):
    mesh = plsc.VectorSubcoreMesh(core_axis_name="c", subcore_axis_name="s")

    @functools.partial(
        pl.kernel,
        mesh=mesh,
        compiler_params=pltpu.CompilerParams(needs_layout_passes=False),
        out_type=jax.ShapeDtypeStruct((NWK, NNP * M), f32),
        scratch_types=(
            [pltpu.VMEM((NNP,), f32)] * (2 * M)
            + [pltpu.VMEM((CS,), f32)] * M
            + [
                pltpu.VMEM((CS,), i32),
                pltpu.VMEM((CS,), i32),
                pltpu.VMEM((NNP * M,), f32),
            ]
        ),
    )
    def k(p0_hbm, p1_hbm, p2_hbm, p3_hbm, p4_hbm, p5_hbm, p6_hbm, p7_hbm,
          e0_hbm, e1_hbm, e2_hbm, e3_hbm, i0_hbm, i1_hbm, out_hbm,
          p0_v, p1_v, p2_v, p3_v, p4_v, p5_v, p6_v, p7_v,
          e0_v, e1_v, e2_v, e3_v, i0_v, i1_v, acc_v):
        p_hbm = [p0_hbm, p1_hbm, p2_hbm, p3_hbm, p4_hbm, p5_hbm, p6_hbm,
                 p7_hbm]
        e_hbm = [e0_hbm, e1_hbm, e2_hbm, e3_hbm]
        p_v = [p0_v, p1_v, p2_v, p3_v, p4_v, p5_v, p6_v, p7_v]
        e_v = [e0_v, e1_v, e2_v, e3_v]

        cid = lax.axis_index("c")
        sid = lax.axis_index("s")
        wid = cid * 16 + sid

        zero16 = jnp.zeros((16,), f32)

        def zero_body(r, c2):
            for c in range(8):
                acc_v[pl.ds(r * 128 + c * 16, 16)] = zero16
            return c2

        lax.fori_loop(0, NNP * M // 128, zero_body, 0)
        for c in range(2 * M):
            pltpu.sync_copy(p_hbm[c], p_v[c])

        def group_body(g, c2):
            off = g * 16
            iv0 = i0_v[pl.ds(off, 16)]
            iv1 = i1_v[pl.ds(off, 16)]
            ps = []
            for c in range(M):
                p = (plsc.load_gather(p_v[c], [iv0])
                     + plsc.load_gather(p_v[c + M], [iv1])
                     + e_v[c][pl.ds(off, 16)])
                ps.append(jnp.where(p >= 0.0, p, p * 0.01))
            mx = jnp.maximum(jnp.maximum(ps[0], ps[1]),
                             jnp.maximum(ps[2], ps[3]))
            es = [jnp.exp(p - mx) for p in ps]
            inv = 1.0 / (es[0] + es[1] + es[2] + es[3])
            n4 = iv0 * M
            for c in range(M):
                plsc.addupdate_scatter(acc_v, [n4 + c], es[c] * inv)
            return c2

        def chunk_body(t, carry):
            base = wid * EPW + t * CS
            pltpu.sync_copy(i0_hbm.at[pl.ds(base, CS)], i0_v)
            pltpu.sync_copy(i1_hbm.at[pl.ds(base, CS)], i1_v)
            for c in range(M):
                pltpu.sync_copy(e_hbm[c].at[pl.ds(base, CS)], e_v[c])
            lax.fori_loop(0, CS // 16, group_body, 0)
            return carry

        lax.fori_loop(0, NCH, chunk_body, 0)

        pltpu.sync_copy(acc_v, out_hbm.at[wid])

    return k(*pcs, *ecs, i0, i1)


def _stage_r_body(acc_ref, out_ref):
    out_ref[...] = jnp.sum(acc_ref[...], axis=0)


def _stage_r(acc):
    grid = 5
    rb = ACCR // grid
    return pl.pallas_call(
        _stage_r_body,
        grid=(grid,),
        in_specs=[pl.BlockSpec((NWK, rb, 128), lambda i: (0, i, 0))],
        out_specs=pl.BlockSpec((rb, 128), lambda i: (i, 0)),
        out_shape=jax.ShapeDtypeStruct((ACCR, 128), f32),
    )(acc)


def _stage_c_body(nf_ref, acc_ref, wn_ref, wm_ref, b_ref, z_ref):
    a = acc_ref[...]
    z_ref[...] = (
        lax.dot_general(nf_ref[...], wn_ref[...], (((1,), (0,)), ((), ())),
                        preferred_element_type=f32)
        + lax.dot_general(a, wm_ref[...], (((1,), (0,)), ((), ())),
                          preferred_element_type=f32)
        + b_ref[...])


def _stage_c(nf, msum, wn, wm, b):
    grid = 5
    nb = 2048
    return pl.pallas_call(
        _stage_c_body,
        grid=(grid,),
        in_specs=[
            pl.BlockSpec((nb, D), lambda i: (i, 0)),
            pl.BlockSpec((nb, M), lambda i: (i, 0)),
            pl.BlockSpec((D, D), lambda i: (0, 0)),
            pl.BlockSpec((M, D), lambda i: (0, 0)),
            pl.BlockSpec((1, D), lambda i: (0, 0)),
        ],
        out_specs=pl.BlockSpec((nb, D), lambda i: (i, 0)),
        out_shape=jax.ShapeDtypeStruct((NN, D), f32),
    )(nf, msum, wn, wm, b)


def kernel(node_features, edge_node_indices, edge_features, fe_W, fe_b, fh_W, fh_b):
    nf = node_features.astype(f32)
    idx0 = jnp.concatenate([edge_node_indices[0].astype(i32),
                            jnp.full((NEP - NE,), NN, i32)])
    idx1 = jnp.concatenate([edge_node_indices[1].astype(i32),
                            jnp.zeros((NEP - NE,), i32)])
    ef = edge_features.astype(f32)

    wcat = jnp.concatenate([fe_W[:D], fe_W[D:2 * D]], axis=1)
    brow = jnp.concatenate([fe_b, jnp.zeros((M,), f32)]).reshape(1, 2 * M)
    wcat_aug = jnp.concatenate([wcat, brow], axis=0)
    nfa = jnp.pad(jnp.concatenate([nf, jnp.ones((NN, 1), f32)], axis=1),
                  ((0, NNP - NN), (0, 0)))
    wcols = [wcat_aug[:, c:c + 1] for c in range(2 * M)]
    w2cols = [fe_W[2 * D:, c:c + 1] for c in range(M)]

    efp = jnp.pad(ef, ((0, NEP - NE), (0, 0)))

    pcs = _stage_a1(nfa, wcols)
    ecs = _stage_a2(efp, w2cols)

    acc32 = _sc_messages(pcs, ecs, idx0, idx1).reshape(NWK, ACCR, 128)
    msum = _stage_r(acc32).reshape(NNP, M)[:NN]

    return _stage_c(nf, msum, fh_W[:D], fh_W[D:], fh_b.reshape(1, D))
